# baseline (speedup 1.0000x reference)
import functools

import jax
import jax.numpy as jnp
from jax import lax
from jax.experimental import pallas as pl
from jax.experimental.pallas import tpu as pltpu
from jax.experimental.pallas import tpu_sc as plsc

B = 16384
AGE_BOUNDARIES = (18., 25., 30., 35., 40., 45., 50., 55., 60., 65.)
N_BUCKETS = 11
THAL_VOCAB = 3
HASH_BUCKETS = 1000
N_COMBO = N_BUCKETS * THAL_VOCAB

_OFF_AGE = 0
_OFF_AB = 1
_OFF_CROSS = 12
_OFF_CA = 1012
_OFF_CHOL = 1013
_OFF_OLDPEAK = 1014
_OFF_SLOPE = 1015
_OFF_EMB = 1016
_OFF_THAL_OH = 1024
_OFF_THALACH = 1027
_OFF_TRESTBPS = 1028

_DENSE_ROWS = (_OFF_AGE, _OFF_CA, _OFF_CHOL, _OFF_OLDPEAK, _OFF_SLOPE,
               _OFF_THALACH, _OFF_TRESTBPS)

_NW = 32
_BPW = B // _NW
_GCHUNK = 128

_TN = (((0,), (0,)), ((), ()))


def _crossed_idx(ab: int, th: int) -> int:
    return (ab * 1000003 + th * 7919) % HASH_BUCKETS


def _build_tables_kernel(w1_ref, emb_ref, b1_ref, t_ref, r_ref):
    e = jax.lax.dot_general(emb_ref[...], w1_ref[_OFF_EMB:_OFF_EMB + 8, :],
                            (((1,), (0,)), ((), ())),
                            preferred_element_type=jnp.float32)
    b1 = b1_ref[...]
    rows = []
    for ab in range(N_BUCKETS):
        for th in range(THAL_VOCAB):
            c = _crossed_idx(ab, th)
            rows.append(w1_ref[_OFF_AB + ab, :] + w1_ref[_OFF_CROSS + c, :]
                        + e[th, :] + w1_ref[_OFF_THAL_OH + th, :] + b1)
    t_ref[...] = jnp.stack(rows, axis=0)
    r_ref[...] = jnp.stack([w1_ref[r, :] for r in _DENSE_ROWS], axis=0)


def _sc_lookup_body(age_hbm, thal_hbm, table_hbm, out_hbm,
                    age_v, thal_v, idx_v, rows_v, sem):
    wid = lax.axis_index("s") * 2 + lax.axis_index("c")
    base = wid * _BPW
    pltpu.sync_copy(age_hbm.at[pl.ds(base, _BPW)], age_v)
    pltpu.sync_copy(thal_hbm.at[pl.ds(base, _BPW)], thal_v)
    for i in range(_BPW // 16):
        a = age_v[pl.ds(i * 16, 16)]
        idx = thal_v[pl.ds(i * 16, 16)]
        for bound in AGE_BOUNDARIES:
            idx = jnp.where(a >= bound, idx + THAL_VOCAB, idx)
        idx_v[pl.ds(i * 16, 16)] = idx
    copies = [
        pltpu.async_copy(table_hbm.at[idx_v.at[pl.ds(j * _GCHUNK, _GCHUNK)]],
                         rows_v.at[pl.ds(j * _GCHUNK, _GCHUNK)], sem)
        for j in range(_BPW // _GCHUNK)
    ]
    for c in copies:
        c.wait()
    pltpu.sync_copy(rows_v, out_hbm.at[pl.ds(base, _BPW)])


@functools.cache
def _sc_lookup():
    return pl.kernel(
        _sc_lookup_body,
        mesh=plsc.VectorSubcoreMesh(core_axis_name="c", subcore_axis_name="s"),
        out_type=jax.ShapeDtypeStruct((B, 128), jnp.float32),
        scratch_types=[
            pltpu.VMEM((_BPW,), jnp.float32),
            pltpu.VMEM((_BPW,), jnp.int32),
            pltpu.VMEM((_BPW,), jnp.int32),
            pltpu.VMEM((_BPW, 128), jnp.float32),
            pltpu.SemaphoreType.DMA,
        ],
    )


def _mlp_kernel(cat_ref, s7_ref, r_ref, w2_ref, b2_ref, w3w_ref, b3_ref,
                out_ref):
    dense = jax.lax.dot_general(s7_ref[...], r_ref[...],
                                (((1,), (0,)), ((), ())),
                                preferred_element_type=jnp.float32)
    h1 = jnp.maximum(cat_ref[...] + dense, 0.0)
    h2 = jax.lax.dot_general(h1, w2_ref[...],
                             (((1,), (0,)), ((), ())),
                             preferred_element_type=jnp.float32)
    h2 = jnp.maximum(h2 + b2_ref[...], 0.0)
    o = jax.lax.dot_general(h2, w3w_ref[...],
                            (((1,), (0,)), ((), ())),
                            preferred_element_type=jnp.float32)
    o = o + b3_ref[...]
    s = 1.0 / (1.0 + jnp.exp(-o))
    out_ref[...] = s[:, :1]


def kernel(age, trestbps, chol, thalach, oldpeak, slope, ca, thal,
           emb_table, W1, b1, W2, b2, W3, b3):
    t33, r7 = pl.pallas_call(
        _build_tables_kernel,
        out_shape=(jax.ShapeDtypeStruct((N_COMBO, 128), jnp.float32),
                   jax.ShapeDtypeStruct((7, 128), jnp.float32)),
    )(W1, emb_table, b1)

    cat = _sc_lookup()(age, thal, t33)

    s7 = jnp.stack([age, ca, chol, oldpeak, slope, thalach, trestbps], axis=1)
    w3_wide = jnp.broadcast_to(W3, (64, 128))

    bb = 4096
    grid = B // bb
    out = pl.pallas_call(
        _mlp_kernel,
        grid=(grid,),
        in_specs=[
            pl.BlockSpec((bb, 128), lambda i: (i, 0)),
            pl.BlockSpec((bb, 7), lambda i: (i, 0)),
            pl.BlockSpec((7, 128), lambda i: (0, 0)),
            pl.BlockSpec((128, 64), lambda i: (0, 0)),
            pl.BlockSpec((1, 64), lambda i: (0, 0)),
            pl.BlockSpec((64, 128), lambda i: (0, 0)),
            pl.BlockSpec((1, 1), lambda i: (0, 0)),
        ],
        out_specs=pl.BlockSpec((bb, 1), lambda i: (i, 0)),
        out_shape=jax.ShapeDtypeStruct((B, 1), jnp.float32),
    )(cat, s7, r7, W2, b2[None, :], w3_wide, b3[None, :])
    return out


# speedup vs baseline: 4.2112x; 4.2112x over previous
import functools

import jax
import jax.numpy as jnp
from jax import lax
from jax.experimental import pallas as pl
from jax.experimental.pallas import tpu as pltpu
from jax.experimental.pallas import tpu_sc as plsc

B = 16384
AGE_BOUNDARIES = (18., 25., 30., 35., 40., 45., 50., 55., 60., 65.)
N_BUCKETS = 11
THAL_VOCAB = 3
HASH_BUCKETS = 1000
N_COMBO = N_BUCKETS * THAL_VOCAB

_OFF_AGE = 0
_OFF_AB = 1
_OFF_CROSS = 12
_OFF_CA = 1012
_OFF_CHOL = 1013
_OFF_OLDPEAK = 1014
_OFF_SLOPE = 1015
_OFF_EMB = 1016
_OFF_THAL_OH = 1024
_OFF_THALACH = 1027
_OFF_TRESTBPS = 1028

_DENSE_ROWS = (_OFF_AGE, _OFF_CA, _OFF_CHOL, _OFF_OLDPEAK, _OFF_SLOPE,
               _OFF_THALACH, _OFF_TRESTBPS)

_NW = 32
_BPW = B // _NW

_TN = (((0,), (0,)), ((), ()))


def _crossed_idx(ab: int, th: int) -> int:
    return (ab * 1000003 + th * 7919) % HASH_BUCKETS


def _sc_idx_body(age_hbm, thal_hbm, out_hbm, age_v, thal_v, idx_v):
    wid = lax.axis_index("s") * 2 + lax.axis_index("c")
    base = wid * _BPW
    pltpu.sync_copy(age_hbm.at[pl.ds(base, _BPW)], age_v)
    pltpu.sync_copy(thal_hbm.at[pl.ds(base, _BPW)], thal_v)
    for i in range(_BPW // 16):
        a = age_v[pl.ds(i * 16, 16)]
        idx = thal_v[pl.ds(i * 16, 16)]
        for bound in AGE_BOUNDARIES:
            idx = jnp.where(a >= bound, idx + THAL_VOCAB, idx)
        idx_v[pl.ds(i * 16, 16)] = idx
    pltpu.sync_copy(idx_v, out_hbm.at[pl.ds(base, _BPW)])


@functools.cache
def _sc_idx():
    return pl.kernel(
        _sc_idx_body,
        mesh=plsc.VectorSubcoreMesh(core_axis_name="c", subcore_axis_name="s"),
        out_type=jax.ShapeDtypeStruct((B,), jnp.int32),
        scratch_types=[
            pltpu.VMEM((_BPW,), jnp.float32),
            pltpu.VMEM((_BPW,), jnp.int32),
            pltpu.VMEM((_BPW,), jnp.int32),
        ],
    )


def _fused_kernel(idx_ref, age_ref, ca_ref, chol_ref, old_ref, slope_ref,
                  tha_ref, tre_ref, w1_ref, emb_ref, b1_ref, w2_ref, b2_ref,
                  w3_ref, b3_ref, out_ref):
    e = jax.lax.dot_general(emb_ref[...], w1_ref[_OFF_EMB:_OFF_EMB + 8, :],
                            (((1,), (0,)), ((), ())),
                            preferred_element_type=jnp.float32)
    b1 = b1_ref[0, :]
    rows = []
    for ab in range(N_BUCKETS):
        for th in range(THAL_VOCAB):
            c = _crossed_idx(ab, th)
            rows.append(w1_ref[_OFF_AB + ab, :] + w1_ref[_OFF_CROSS + c, :]
                        + e[th, :] + w1_ref[_OFF_THAL_OH + th, :] + b1)
    t33 = jnp.stack(rows, axis=0)
    r7 = jnp.stack([w1_ref[r, :] for r in _DENSE_ROWS], axis=0)

    idx = idx_ref[...]
    combos = jax.lax.broadcasted_iota(jnp.int32, (N_COMBO, idx.shape[1]), 0)
    onehot_t = (combos == idx).astype(jnp.float32)
    s7_t = jnp.concatenate([age_ref[...], ca_ref[...], chol_ref[...],
                            old_ref[...], slope_ref[...], tha_ref[...],
                            tre_ref[...]], axis=0)
    cat_t = jax.lax.dot_general(t33, onehot_t, _TN,
                                preferred_element_type=jnp.float32)
    dense_t = jax.lax.dot_general(r7, s7_t, _TN,
                                  preferred_element_type=jnp.float32)
    h1_t = jnp.maximum(cat_t + dense_t, 0.0)
    h2_t = jax.lax.dot_general(w2_ref[...], h1_t, _TN,
                               preferred_element_type=jnp.float32)
    h2_t = jnp.maximum(h2_t + b2_ref[...], 0.0)
    o_t = jax.lax.dot_general(w3_ref[...], h2_t, _TN,
                              preferred_element_type=jnp.float32)
    o_t = o_t + b3_ref[...]
    out_ref[...] = 1.0 / (1.0 + jnp.exp(-o_t))


def kernel(age, trestbps, chol, thalach, oldpeak, slope, ca, thal,
           emb_table, W1, b1, W2, b2, W3, b3):
    idx = _sc_idx()(age, thal)

    bb = 4096
    grid = B // bb
    row = pl.BlockSpec((1, bb), lambda i: (0, i))
    full = lambda a, b: pl.BlockSpec((a, b), lambda i: (0, 0))
    out_t = pl.pallas_call(
        _fused_kernel,
        grid=(grid,),
        in_specs=[row, row, row, row, row, row, row, row,
                  full(1029, 128), full(THAL_VOCAB, 8), full(1, 128),
                  full(128, 64), full(64, 1), full(64, 1), full(1, 1)],
        out_specs=row,
        out_shape=jax.ShapeDtypeStruct((1, B), jnp.float32),
    )(idx[None, :], age[None, :], ca[None, :], chol[None, :],
      oldpeak[None, :], slope[None, :], thalach[None, :], trestbps[None, :],
      W1, emb_table, b1[None, :], W2, b2[:, None], W3, b3[:, None])
    return out_t.reshape(B, 1)


# speedup vs baseline: 15.0524x; 3.5744x over previous
import jax
import jax.numpy as jnp
from jax.experimental import pallas as pl

B = 16384
AGE_BOUNDARIES = (18., 25., 30., 35., 40., 45., 50., 55., 60., 65.)
N_BUCKETS = 11
THAL_VOCAB = 3
HASH_BUCKETS = 1000
N_COMBO = N_BUCKETS * THAL_VOCAB

_OFF_AGE = 0
_OFF_AB = 1
_OFF_CROSS = 12
_OFF_CA = 1012
_OFF_CHOL = 1013
_OFF_OLDPEAK = 1014
_OFF_SLOPE = 1015
_OFF_EMB = 1016
_OFF_THAL_OH = 1024
_OFF_THALACH = 1027
_OFF_TRESTBPS = 1028

_DENSE_ROWS = (_OFF_AGE, _OFF_CA, _OFF_CHOL, _OFF_OLDPEAK, _OFF_SLOPE,
               _OFF_THALACH, _OFF_TRESTBPS)

_TN = (((0,), (0,)), ((), ()))


def _crossed_idx(ab: int, th: int) -> int:
    return (ab * 1000003 + th * 7919) % HASH_BUCKETS


def _fused_kernel(age_ref, thal_ref, ca_ref, chol_ref, old_ref, slope_ref,
                  tha_ref, tre_ref, w1_ref, emb_ref, b1_ref, w2_ref, b2_ref,
                  w3_ref, b3_ref, out_ref):
    e = jax.lax.dot_general(emb_ref[...], w1_ref[_OFF_EMB:_OFF_EMB + 8, :],
                            (((1,), (0,)), ((), ())),
                            preferred_element_type=jnp.float32)
    b1 = b1_ref[0, :]
    rows = []
    for ab in range(N_BUCKETS):
        for th in range(THAL_VOCAB):
            c = _crossed_idx(ab, th)
            rows.append(w1_ref[_OFF_AB + ab, :] + w1_ref[_OFF_CROSS + c, :]
                        + e[th, :] + w1_ref[_OFF_THAL_OH + th, :] + b1)
    for r in _DENSE_ROWS:
        rows.append(w1_ref[r, :])
    t40 = jnp.stack(rows, axis=0)

    age = age_ref[...]
    ab_i = jnp.zeros_like(age, dtype=jnp.int32)
    for bound in AGE_BOUNDARIES:
        ab_i = ab_i + (age >= bound).astype(jnp.int32)
    idx = ab_i * THAL_VOCAB + thal_ref[...]
    combos = jax.lax.broadcasted_iota(jnp.int32, (N_COMBO, idx.shape[1]), 0)
    onehot_t = (combos == idx).astype(jnp.float32)
    x40 = jnp.concatenate([onehot_t, age, ca_ref[...], chol_ref[...],
                           old_ref[...], slope_ref[...], tha_ref[...],
                           tre_ref[...]], axis=0)
    h1_t = jnp.maximum(jax.lax.dot_general(
        t40, x40, _TN, preferred_element_type=jnp.float32), 0.0)
    h2_t = jax.lax.dot_general(w2_ref[...], h1_t, _TN,
                               preferred_element_type=jnp.float32)
    h2_t = jnp.maximum(h2_t + b2_ref[...], 0.0)
    o_t = jax.lax.dot_general(w3_ref[...], h2_t, _TN,
                              preferred_element_type=jnp.float32)
    o_t = o_t + b3_ref[...]
    out_ref[...] = 1.0 / (1.0 + jnp.exp(-o_t))


def kernel(age, trestbps, chol, thalach, oldpeak, slope, ca, thal,
           emb_table, W1, b1, W2, b2, W3, b3):
    bb = 16384
    grid = B // bb
    row = pl.BlockSpec((1, bb), lambda i: (0, i))
    full = lambda a, b: pl.BlockSpec((a, b), lambda i: (0, 0))
    out_t = pl.pallas_call(
        _fused_kernel,
        grid=(grid,),
        in_specs=[row, row, row, row, row, row, row, row,
                  full(1029, 128), full(THAL_VOCAB, 8), full(1, 128),
                  full(128, 64), full(64, 1), full(64, 1), full(1, 1)],
        out_specs=row,
        out_shape=jax.ShapeDtypeStruct((1, B), jnp.float32),
    )(age[None, :], thal[None, :], ca[None, :], chol[None, :],
      oldpeak[None, :], slope[None, :], thalach[None, :], trestbps[None, :],
      W1, emb_table, b1[None, :], W2, b2[:, None], W3, b3[:, None])
    return out_t.reshape(B, 1)
